# baseline (speedup 1.0000x reference)
import math

import jax
import jax.numpy as jnp
from jax.experimental import pallas as pl
from jax.experimental.pallas import tpu as pltpu

_EMBED_DIM = 64
_LOG10000 = math.log(10000.0)


def _body(var_ref, out_ref, acc_ref):
    T, D, E = acc_ref.shape

    @pl.when(pl.program_id(0) == 0)
    def _compute_sum():
        pos = jax.lax.broadcasted_iota(jnp.int32, (T, E), 0).astype(jnp.float32)
        e_idx = jax.lax.broadcasted_iota(jnp.int32, (T, E), 1)
        k = (e_idx // 2).astype(jnp.float32)
        freq = jnp.exp(k * (-2.0 * _LOG10000 / E))
        angle = pos * freq
        pe = jnp.where(e_idx % 2 == 0, jnp.sin(angle), jnp.cos(angle))
        acc_ref[...] = var_ref[...][None, :, :] + pe[:, None, :]

    out_ref[0] = acc_ref[...]


def kernel(x, var_table):
    B, T, D = x.shape
    E = _EMBED_DIM
    return pl.pallas_call(
        _body,
        grid=(B,),
        in_specs=[pl.BlockSpec((D, E), lambda b: (0, 0))],
        out_specs=pl.BlockSpec((1, T, D, E), lambda b: (b, 0, 0, 0)),
        out_shape=jax.ShapeDtypeStruct((B, T, D, E), jnp.float32),
        scratch_shapes=[pltpu.VMEM((T, D, E), jnp.float32)],
    )(var_table)
